# baseline (speedup 1.0000x reference)
import jax
import jax.numpy as jnp
from jax import lax
from jax.experimental import pallas as pl
from jax.experimental.pallas import tpu as pltpu
from jax.experimental.pallas import tpu_sc as plsc

_BLOCK_ROWS = 4096


def _tc_copy_body(in_ref, out_ref):
    out_ref[...] = in_ref[...]


def _make_sc_copy(shape, dtype):
    info = plsc.get_sparse_core_info()
    nc, ns = info.num_cores, info.num_subcores
    nw = nc * ns
    rows = shape[0] // nw
    mesh = plsc.VectorSubcoreMesh(core_axis_name="c", subcore_axis_name="s")

    def body(in_hbm, out_hbm):
        wid = lax.axis_index("s") * nc + lax.axis_index("c")
        base = wid * rows
        pltpu.sync_copy(
            in_hbm.at[pl.ds(base, rows)],
            out_hbm.at[pl.ds(base, rows)],
        )

    return pl.kernel(
        body,
        out_type=jax.ShapeDtypeStruct(shape, dtype),
        mesh=mesh,
    )


def kernel(input0, input1, input2, input3):
    M, B = input0.shape
    spec = pl.BlockSpec((_BLOCK_ROWS, B), lambda i: (i, 0))
    out0 = pl.pallas_call(
        _tc_copy_body,
        grid=(M // _BLOCK_ROWS,),
        in_specs=[spec],
        out_specs=spec,
        out_shape=jax.ShapeDtypeStruct((M, B), input0.dtype),
        compiler_params=pltpu.CompilerParams(
            dimension_semantics=("parallel",),
        ),
    )(input0)
    out1 = _make_sc_copy((M, B), input1.dtype)(input1)
    return (out0, out1)
